# baseline (speedup 1.0000x reference)
import functools

import jax
import jax.numpy as jnp
from jax import lax
from jax.experimental import pallas as pl
from jax.experimental.pallas import tpu as pltpu
from jax.experimental.pallas import tpu_sc as plsc

NC = 2
NS = 16
NW = NC * NS
CHUNK = 128
NBUF = 2


@functools.partial(jax.jit, static_argnames=())
def _embed_lookup(table, idx3):
    _, D = table.shape
    nw, nchunks, chunk = idx3.shape
    n = nw * nchunks * chunk
    rows_per_w = nchunks * chunk
    mesh = plsc.VectorSubcoreMesh(core_axis_name="c", subcore_axis_name="s")

    @functools.partial(
        pl.kernel,
        mesh=mesh,
        compiler_params=pltpu.CompilerParams(use_tc_tiling_on_sc=False),
        out_type=jax.ShapeDtypeStruct((n, D), jnp.float32),
        scratch_types=[
            pltpu.VMEM((nchunks, chunk), jnp.int32),
            pltpu.VMEM((NBUF, chunk, D), jnp.float32),
            pltpu.SemaphoreType.DMA,
            pltpu.SemaphoreType.DMA,
        ],
    )
    def k(table_hbm, idx_hbm, out_hbm, idx_v, rows_v, sem0, sem1):
        sems = [sem0, sem1]
        wid = lax.axis_index("s") * NC + lax.axis_index("c")
        base = wid * rows_per_w
        pltpu.sync_copy(idx_hbm.at[wid], idx_v)

        def gather_start(chunk_i, b):
            pltpu.async_copy(table_hbm.at[idx_v.at[chunk_i]], rows_v.at[b],
                             sems[b])

        def gather_wait(b):
            pltpu.make_async_copy(table_hbm.at[pl.ds(0, chunk)], rows_v.at[b],
                                  sems[b]).wait()

        def write_out(chunk_i, b):
            pltpu.sync_copy(rows_v.at[b],
                            out_hbm.at[pl.ds(base + chunk_i * chunk, chunk)])

        for b in range(NBUF):
            gather_start(b, b)

        def body(i, carry):
            g = i * NBUF
            for b in range(NBUF):
                ci = g + b
                gather_wait(b)
                write_out(ci, b)
                gather_start(ci + NBUF, b)
            return carry

        lax.fori_loop(0, (nchunks - NBUF) // NBUF, body, 0)

        for b in range(NBUF):
            ci = nchunks - NBUF + b
            gather_wait(b)
            write_out(ci, b)

    return k(table, idx3)


def kernel(x, embed_weight):
    batch, hist = x.shape
    _, d = embed_weight.shape
    n = batch * hist
    assert n % (NW * CHUNK) == 0
    nchunks = n // (NW * CHUNK)
    idx3 = x.astype(jnp.int32).reshape(NW, nchunks, CHUNK)
    out = _embed_lookup(embed_weight, idx3)
    return out.reshape(batch, hist, d)


# speedup vs baseline: 1.0101x; 1.0101x over previous
import functools

import jax
import jax.numpy as jnp
from jax import lax
from jax.experimental import pallas as pl
from jax.experimental.pallas import tpu as pltpu
from jax.experimental.pallas import tpu_sc as plsc

NC = 2
NS = 16
NW = NC * NS
CHUNK = 256
NBUF = 5


@functools.partial(jax.jit, static_argnames=())
def _embed_lookup(table, idx3):
    _, D = table.shape
    nw, nchunks, chunk = idx3.shape
    n = nw * nchunks * chunk
    rows_per_w = nchunks * chunk
    mesh = plsc.VectorSubcoreMesh(core_axis_name="c", subcore_axis_name="s")

    @functools.partial(
        pl.kernel,
        mesh=mesh,
        compiler_params=pltpu.CompilerParams(use_tc_tiling_on_sc=False),
        out_type=jax.ShapeDtypeStruct((n, D), jnp.float32),
        scratch_types=[
            pltpu.VMEM((nchunks, chunk), jnp.int32),
            pltpu.VMEM((NBUF, chunk, D), jnp.float32),
        ] + [pltpu.SemaphoreType.DMA] * NBUF,
    )
    def k(table_hbm, idx_hbm, out_hbm, idx_v, rows_v, *sems):
        wid = lax.axis_index("s") * NC + lax.axis_index("c")
        base = wid * rows_per_w
        pltpu.sync_copy(idx_hbm.at[wid], idx_v)

        def gather_start(chunk_i, b):
            pltpu.async_copy(table_hbm.at[idx_v.at[chunk_i]], rows_v.at[b],
                             sems[b])

        def gather_wait(b):
            pltpu.make_async_copy(table_hbm.at[pl.ds(0, chunk)], rows_v.at[b],
                                  sems[b]).wait()

        def write_out(chunk_i, b):
            pltpu.sync_copy(rows_v.at[b],
                            out_hbm.at[pl.ds(base + chunk_i * chunk, chunk)])

        for b in range(NBUF):
            gather_start(b, b)

        def body(i, carry):
            g = i * NBUF
            for b in range(NBUF):
                ci = g + b
                gather_wait(b)
                write_out(ci, b)
                gather_start(ci + NBUF, b)
            return carry

        lax.fori_loop(0, (nchunks - NBUF) // NBUF, body, 0)

        for b in range(NBUF):
            ci = nchunks - NBUF + b
            gather_wait(b)
            write_out(ci, b)

    return k(table, idx3)


def kernel(x, embed_weight):
    batch, hist = x.shape
    _, d = embed_weight.shape
    n = batch * hist
    assert n % (NW * CHUNK) == 0
    nchunks = n // (NW * CHUNK)
    idx3 = x.astype(jnp.int32).reshape(NW, nchunks, CHUNK)
    out = _embed_lookup(embed_weight, idx3)
    return out.reshape(batch, hist, d)
